# baseline (speedup 1.0000x reference)
import functools

import jax
import jax.numpy as jnp
import numpy as np
from jax.experimental import pallas as pl
from jax.experimental.pallas import tpu as pltpu


def _pool_conv_kernel(*refs):
    *x_refs, p_ref, w_ref, o_ref = refs
    p = p_ref[...]
    pooled = jnp.concatenate(
        [jnp.dot(x_ref[0].astype(jnp.bfloat16), p,
                 preferred_element_type=jnp.float32) for x_ref in x_refs],
        axis=0)
    out = jnp.dot(w_ref[...], pooled.astype(jnp.bfloat16),
                  preferred_element_type=jnp.float32)
    o_ref[0] = out.astype(o_ref.dtype)


@functools.lru_cache(maxsize=32)
def _pool_operator(th, scale, w_in, w_out):
    L, S = th * scale * w_in, th * w_out
    li, si = np.arange(L), np.arange(S)
    row_hit = (li[:, None] // (scale * w_in)) == (si[None, :] // w_out)
    col_hit = (li[:, None] % w_in) // scale == (si[None, :] % w_out)
    return ((row_hit & col_hit).astype(np.float32) / (scale * scale)).astype(
        jnp.bfloat16)


def _pick_th(hs, ws, w_in, scale, c, itemsize):
    vmem_budget = 36 * 2**20
    best = None
    for th in range(1, hs + 1):
        if hs % th:
            continue
        L, S = th * scale * w_in, th * ws
        if th != hs and (L % 128 or S % 128):
            continue
        need = 2 * itemsize * c * L + 2 * c * L + 2 * L * S
        if need <= vmem_budget or best is None:
            best = th
        if need > vmem_budget and best is not None:
            break
    return best


def _run_pool_conv(x, w2d, *, scale, nsplit=2):
    N, C, H, W = x.shape
    Hs, Ws = H // scale, W // scale
    C_out = w2d.shape[0]
    th = _pick_th(Hs, Ws, W, scale, C, x.dtype.itemsize)
    L, S = th * scale * W, th * Ws
    p_mat = jnp.asarray(_pool_operator(th, scale, W, Ws))
    x_flat = x.reshape(N, C, H * W)

    while C % nsplit or (C // nsplit) % 8:
        nsplit -= 1
    Ck = C // nsplit

    grid = (N, Hs // th)
    flops = grid[0] * grid[1] * 2 * (C * L * S + C_out * C * S)
    bytes_accessed = (x_flat.size * x_flat.dtype.itemsize
                      + N * C_out * Hs * Ws * x_flat.dtype.itemsize
                      + p_mat.size * 2 + w2d.size * 2)

    def _x_spec(k):
        return pl.BlockSpec((1, Ck, L), lambda n, h: (n, k, h))

    out_flat = pl.pallas_call(
        _pool_conv_kernel,
        out_shape=jax.ShapeDtypeStruct((N, C_out, Hs * Ws), x.dtype),
        grid=grid,
        in_specs=[_x_spec(k) for k in range(nsplit)] + [
            pl.BlockSpec((L, S), lambda n, h: (0, 0)),
            pl.BlockSpec((C_out, C), lambda n, h: (0, 0)),
        ],
        out_specs=pl.BlockSpec((1, C_out, S), lambda n, h: (n, 0, h)),
        compiler_params=pltpu.CompilerParams(
            dimension_semantics=("parallel", "parallel"),
            vmem_limit_bytes=56 * 2**20,
        ),
        cost_estimate=pl.CostEstimate(flops=int(flops), transcendentals=0,
                                      bytes_accessed=int(bytes_accessed)),
    )(*([x_flat] * nsplit), p_mat, w2d)
    return out_flat.reshape(N, C_out, Hs, Ws)


def kernel(hidden_states, weight, *, scale=8):
    five_d = hidden_states.ndim == 5
    if five_d:
        B, F, C, H, W = hidden_states.shape
        x = hidden_states.reshape(B * F, C, H, W)
    else:
        x = hidden_states
    C_out, C_in = weight.shape[0], weight.shape[1]
    w2d = weight.reshape(C_out, C_in).astype(jnp.bfloat16)
    out = _run_pool_conv(x, w2d, scale=scale)
    if five_d:
        out = out.reshape(B, F, C_out, out.shape[-2], out.shape[-1])
    return out


# speedup vs baseline: 2.8594x; 2.8594x over previous
import functools

import jax
import jax.numpy as jnp
import numpy as np
from jax.experimental import pallas as pl
from jax.experimental.pallas import tpu as pltpu


def _make_pool_kernel(scale):
    def _pool(x_ref, pw_ref, o_ref):
        x = x_ref[0]
        Cb, H, W = x.shape
        Ws = pw_ref.shape[1]
        Hs = H // scale
        y = jnp.dot(x.reshape(Cb * H, W), pw_ref[...],
                    preferred_element_type=jnp.float32)
        pooled = jnp.sum(y.reshape(Cb * Hs, scale, Ws), axis=1)
        o_ref[0] = pooled.reshape(Cb, Hs, Ws).astype(o_ref.dtype)

    return _pool


def _conv_kernel(p_ref, w_ref, o_ref):
    o_ref[0] = jnp.dot(w_ref[...], p_ref[0],
                       preferred_element_type=jnp.float32).astype(o_ref.dtype)


@functools.lru_cache(maxsize=32)
def _col_pool_operator(w_in, scale):
    ws = w_in // scale
    hit = (np.arange(w_in)[:, None] // scale) == np.arange(ws)[None, :]
    return hit.astype(np.float32) / float(scale * scale)


def _pick_cb(c, h, w_in, itemsize, target_bytes=8 * 2**20):
    img = h * w_in * itemsize
    best = 1
    for cb in range(1, c + 1):
        if c % cb == 0 and (cb * img <= target_bytes or best == 1):
            best = cb
    return best


def _run_pool_conv(x, w2d, *, scale):
    N, C, H, W = x.shape
    Hs, Ws = H // scale, W // scale
    C_out = w2d.shape[0]
    Cb = _pick_cb(C, H, W, x.dtype.itemsize)
    pw = jnp.asarray(_col_pool_operator(W, scale))

    pooled = pl.pallas_call(
        _make_pool_kernel(scale),
        out_shape=jax.ShapeDtypeStruct((N, C, Hs, Ws), x.dtype),
        grid=(N, C // Cb),
        in_specs=[
            pl.BlockSpec((1, Cb, H, W), lambda n, c: (n, c, 0, 0)),
            pl.BlockSpec((W, Ws), lambda n, c: (0, 0)),
        ],
        out_specs=pl.BlockSpec((1, Cb, Hs, Ws), lambda n, c: (n, c, 0, 0)),
        compiler_params=pltpu.CompilerParams(
            dimension_semantics=("parallel", "parallel"),
            vmem_limit_bytes=48 * 2**20,
        ),
        cost_estimate=pl.CostEstimate(
            flops=int(2 * N * C * H * W * Ws + N * C * Hs * Ws * scale),
            transcendentals=0,
            bytes_accessed=int(x.size * 4 + N * C * Hs * Ws * 4),
        ),
    )(x, pw)

    out = pl.pallas_call(
        _conv_kernel,
        out_shape=jax.ShapeDtypeStruct((N, C_out, Hs * Ws), x.dtype),
        grid=(N,),
        in_specs=[
            pl.BlockSpec((1, C, Hs * Ws), lambda n: (n, 0, 0)),
            pl.BlockSpec((C_out, C), lambda n: (0, 0)),
        ],
        out_specs=pl.BlockSpec((1, C_out, Hs * Ws), lambda n: (n, 0, 0)),
        compiler_params=pltpu.CompilerParams(
            dimension_semantics=("parallel",),
            vmem_limit_bytes=32 * 2**20,
        ),
        cost_estimate=pl.CostEstimate(
            flops=int(2 * N * C_out * C * Hs * Ws), transcendentals=0,
            bytes_accessed=int(N * (C + C_out) * Hs * Ws * 4),
        ),
    )(pooled.reshape(N, C, Hs * Ws), w2d)
    return out.reshape(N, C_out, Hs, Ws)


def kernel(hidden_states, weight, *, scale=8):
    five_d = hidden_states.ndim == 5
    if five_d:
        B, F, C, H, W = hidden_states.shape
        x = hidden_states.reshape(B * F, C, H, W)
    else:
        x = hidden_states
    C_out, C_in = weight.shape[0], weight.shape[1]
    w2d = weight.reshape(C_out, C_in).astype(x.dtype)
    out = _run_pool_conv(x, w2d, scale=scale)
    if five_d:
        out = out.reshape(B, F, C_out, out.shape[-2], out.shape[-1])
    return out
